# baseline (speedup 1.0000x reference)
import functools

import jax
import jax.numpy as jnp
from jax import lax
from jax.experimental import pallas as pl
from jax.experimental.pallas import tpu as pltpu
from jax.experimental.pallas import tpu_sc as plsc

N_NODES = 10000
N_EDGES = 320000
D = 128

NC = 2
NS = 16
NW = NC * NS
NPT = 320
NPAD = NW * NPT

CHUNK = 4000
NCHUNKS = N_EDGES // CHUNK
GB = 128

BM = 2560


def _seg_max_body(y_hbm, row_hbm, col_hbm, out_hbm,
                  rowbuf, colbuf, mrow, mcol, gbuf, acc, sem):
    cid = lax.axis_index("c")
    sid = lax.axis_index("s")
    wid = sid * NC + cid
    base = wid * NPT

    neg = jnp.full((16,), -jnp.inf, dtype=jnp.float32)

    def init_row(i, carry):
        for d in range(D // 16):
            acc[i, pl.ds(d * 16, 16)] = neg
        return carry

    lax.fori_loop(0, NPT, init_row, 0)

    def chunk_body(ci, carry):
        off = ci * CHUNK
        pltpu.sync_copy(row_hbm.at[pl.ds(off, CHUNK)], rowbuf)
        pltpu.sync_copy(col_hbm.at[pl.ds(off, CHUNK)], colbuf)

        def filt(i, cnt):
            cv = colbuf[pl.ds(i * 16, 16)]
            rv = rowbuf[pl.ds(i * 16, 16)]
            msk = (cv >= base) & (cv < base + NPT)
            mi = lax.select(msk, jnp.ones((16,), jnp.int32), jnp.zeros((16,), jnp.int32))
            cum = plsc.cumsum(mi)
            p = cnt + cum - 1
            plsc.store_scatter(mcol, [p], cv - base, mask=msk)
            plsc.store_scatter(mrow, [p], rv, mask=msk)
            return cnt + cum[15]

        n = lax.fori_loop(0, CHUNK // 16, filt, jnp.int32(0))

        zeros16 = jnp.zeros((16,), dtype=jnp.int32)
        for j in range(GB // 16):
            mrow[pl.ds(n + j * 16, 16)] = zeros16

        nb = (n + GB - 1) // GB

        def batch(b, carry2):
            pltpu.async_copy(y_hbm.at[mrow.at[pl.ds(b * GB, GB)]], gbuf, sem).wait()
            ecount = jnp.minimum(jnp.int32(GB), n - b * GB)

            def edge(i, carry3):
                c = mcol[pl.ds(b * GB + i, 16)][0]
                for d in range(D // 16):
                    sl = pl.ds(d * 16, 16)
                    acc[c, sl] = jnp.maximum(acc[c, sl], gbuf[i, sl])
                return carry3

            lax.fori_loop(0, ecount, edge, 0)
            return carry2

        lax.fori_loop(0, nb, batch, 0)
        return carry

    lax.fori_loop(0, NCHUNKS, chunk_body, jnp.int32(0))

    pltpu.sync_copy(acc, out_hbm.at[pl.ds(base, NPT)])


_seg_max = functools.partial(
    pl.kernel,
    out_type=jax.ShapeDtypeStruct((NPAD, D), jnp.float32),
    mesh=plsc.VectorSubcoreMesh(
        core_axis_name="c", subcore_axis_name="s", num_cores=NC, num_subcores=NS
    ),
    scratch_types=[
        pltpu.VMEM((CHUNK,), jnp.int32),
        pltpu.VMEM((CHUNK,), jnp.int32),
        pltpu.VMEM((CHUNK + GB,), jnp.int32),
        pltpu.VMEM((CHUNK + 16,), jnp.int32),
        pltpu.VMEM((GB, D), jnp.float32),
        pltpu.VMEM((NPT, D), jnp.float32),
        pltpu.SemaphoreType.DMA,
    ],
    compiler_params=pltpu.CompilerParams(needs_layout_passes=False),
)(_seg_max_body)


def _mm_theta_body(x_ref, w_ref, o_ref):
    o_ref[...] = lax.dot_general(
        x_ref[...], w_ref[...], (((1,), (0,)), ((), ())),
        preferred_element_type=jnp.float32,
    )


def _final_body(m_ref, y_ref, w_ref, o_ref):
    m = m_ref[...]
    md = jnp.where(jnp.isfinite(m), m - y_ref[...], 0.0)
    o_ref[...] = jnp.maximum(
        lax.dot_general(md, w_ref[...], (((1,), (0,)), ((), ())),
                        preferred_element_type=jnp.float32),
        0.0,
    )


def kernel(x, edges, W_theta, W_phi):
    xp = jnp.pad(x, ((0, NPAD - N_NODES), (0, 0)))
    row = edges[0]
    col = edges[1]

    y = pl.pallas_call(
        _mm_theta_body,
        grid=(NPAD // BM,),
        in_specs=[
            pl.BlockSpec((BM, D), lambda i: (i, 0)),
            pl.BlockSpec((D, D), lambda i: (0, 0)),
        ],
        out_specs=pl.BlockSpec((BM, D), lambda i: (i, 0)),
        out_shape=jax.ShapeDtypeStruct((NPAD, D), jnp.float32),
    )(xp, W_theta.T)

    m = _seg_max(y, row, col)

    out = pl.pallas_call(
        _final_body,
        grid=(NPAD // BM,),
        in_specs=[
            pl.BlockSpec((BM, D), lambda i: (i, 0)),
            pl.BlockSpec((BM, D), lambda i: (i, 0)),
            pl.BlockSpec((D, D), lambda i: (0, 0)),
        ],
        out_specs=pl.BlockSpec((BM, D), lambda i: (i, 0)),
        out_shape=jax.ShapeDtypeStruct((NPAD, D), jnp.float32),
    )(m, y, W_phi.T)

    return out[:N_NODES]


# speedup vs baseline: 4.5352x; 4.5352x over previous
import functools

import jax
import jax.numpy as jnp
from jax import lax
from jax.experimental import pallas as pl
from jax.experimental.pallas import tpu as pltpu
from jax.experimental.pallas import tpu_sc as plsc

N_NODES = 10000
N_EDGES = 320000
D = 128

NC = 2
NS = 16
NW = NC * NS
NPASS = 2
NPT = 160
NPAD = NW * NPT * NPASS

CHUNK = 4000
NCHUNKS = N_EDGES // CHUNK
GB = 64

BM = 2560


def _seg_max_body(y_hbm, row_hbm, col_hbm, out_hbm,
                  rowbuf, colbuf, mrow, mcol, gbuf, acc, yshared, sem):
    cid = lax.axis_index("c")
    sid = lax.axis_index("s")
    wid = sid * NC + cid

    ystripe = NPAD // NS
    for k in range(ystripe // GB):
        r = sid * ystripe + k * GB
        pltpu.sync_copy(y_hbm.at[pl.ds(r, GB)], gbuf)
        pltpu.sync_copy(gbuf, yshared.at[pl.ds(r, GB)])

    plsc.subcore_barrier()

    neg = jnp.full((16,), -jnp.inf, dtype=jnp.float32)
    zeros16 = jnp.zeros((16,), dtype=jnp.int32)

    for p in range(NPASS):
        base = p * (NW * NPT) + wid * NPT

        def init_row(i, carry):
            for d in range(D // 16):
                acc[i, pl.ds(d * 16, 16)] = neg
            return carry

        lax.fori_loop(0, NPT, init_row, 0)

        def chunk_body(ci, carry):
            off = ci * CHUNK
            pltpu.sync_copy(row_hbm.at[pl.ds(off, CHUNK)], rowbuf)
            pltpu.sync_copy(col_hbm.at[pl.ds(off, CHUNK)], colbuf)

            def filt(i, cnt):
                cv = colbuf[pl.ds(i * 16, 16)]
                rv = rowbuf[pl.ds(i * 16, 16)]
                msk = (cv >= base) & (cv < base + NPT)
                mi = lax.select(msk, jnp.ones((16,), jnp.int32),
                                jnp.zeros((16,), jnp.int32))
                cum = plsc.cumsum(mi)
                pos = cnt + cum - 1
                plsc.store_scatter(mcol, [pos], cv - base, mask=msk)
                plsc.store_scatter(mrow, [pos], rv, mask=msk)
                return cnt + cum[15]

            n = lax.fori_loop(0, CHUNK // 16, filt, jnp.int32(0))

            for j in range(GB // 16):
                mrow[pl.ds(n + j * 16, 16)] = zeros16

            nb = (n + GB - 1) // GB

            def batch(b, carry2):
                pltpu.async_copy(
                    yshared.at[mrow.at[pl.ds(b * GB, GB)]], gbuf, sem
                ).wait()
                ecount = jnp.minimum(jnp.int32(GB), n - b * GB)

                def edge(i, carry3):
                    c = mcol[pl.ds(b * GB + i, 16)][0]
                    for d in range(D // 16):
                        sl = pl.ds(d * 16, 16)
                        acc[c, sl] = jnp.maximum(acc[c, sl], gbuf[i, sl])
                    return carry3

                lax.fori_loop(0, ecount, edge, 0)
                return carry2

            lax.fori_loop(0, nb, batch, 0)
            return carry

        lax.fori_loop(0, NCHUNKS, chunk_body, jnp.int32(0))

        pltpu.sync_copy(acc, out_hbm.at[pl.ds(base, NPT)])


_seg_max = functools.partial(
    pl.kernel,
    out_type=jax.ShapeDtypeStruct((NPAD, D), jnp.float32),
    mesh=plsc.VectorSubcoreMesh(
        core_axis_name="c", subcore_axis_name="s", num_cores=NC, num_subcores=NS
    ),
    scratch_types=[
        pltpu.VMEM((CHUNK,), jnp.int32),
        pltpu.VMEM((CHUNK,), jnp.int32),
        pltpu.VMEM((CHUNK + GB,), jnp.int32),
        pltpu.VMEM((CHUNK + 16,), jnp.int32),
        pltpu.VMEM((GB, D), jnp.float32),
        pltpu.VMEM((NPT, D), jnp.float32),
        pltpu.VMEM_SHARED((NPAD, D), jnp.float32),
        pltpu.SemaphoreType.DMA,
    ],
    compiler_params=pltpu.CompilerParams(needs_layout_passes=False),
)(_seg_max_body)


def _mm_theta_body(x_ref, w_ref, o_ref):
    o_ref[...] = lax.dot_general(
        x_ref[...], w_ref[...], (((1,), (0,)), ((), ())),
        preferred_element_type=jnp.float32,
    )


def _final_body(m_ref, y_ref, w_ref, o_ref):
    m = m_ref[...]
    md = jnp.where(jnp.isfinite(m), m - y_ref[...], 0.0)
    o_ref[...] = jnp.maximum(
        lax.dot_general(md, w_ref[...], (((1,), (0,)), ((), ())),
                        preferred_element_type=jnp.float32),
        0.0,
    )


def kernel(x, edges, W_theta, W_phi):
    xp = jnp.pad(x, ((0, NPAD - N_NODES), (0, 0)))
    row = edges[0]
    col = edges[1]

    y = pl.pallas_call(
        _mm_theta_body,
        grid=(NPAD // BM,),
        in_specs=[
            pl.BlockSpec((BM, D), lambda i: (i, 0)),
            pl.BlockSpec((D, D), lambda i: (0, 0)),
        ],
        out_specs=pl.BlockSpec((BM, D), lambda i: (i, 0)),
        out_shape=jax.ShapeDtypeStruct((NPAD, D), jnp.float32),
    )(xp, W_theta.T)

    m = _seg_max(y, row, col)

    out = pl.pallas_call(
        _final_body,
        grid=(NPAD // BM,),
        in_specs=[
            pl.BlockSpec((BM, D), lambda i: (i, 0)),
            pl.BlockSpec((BM, D), lambda i: (i, 0)),
            pl.BlockSpec((D, D), lambda i: (0, 0)),
        ],
        out_specs=pl.BlockSpec((BM, D), lambda i: (i, 0)),
        out_shape=jax.ShapeDtypeStruct((NPAD, D), jnp.float32),
    )(m, y, W_phi.T)

    return out[:N_NODES]


# speedup vs baseline: 5.0032x; 1.1032x over previous
import functools

import jax
import jax.numpy as jnp
from jax import lax
from jax.experimental import pallas as pl
from jax.experimental.pallas import tpu as pltpu
from jax.experimental.pallas import tpu_sc as plsc

N_NODES = 10000
N_EDGES = 320000
D = 128

NC = 2
NS = 16
NW = NC * NS
NPASS = 2
NPT = 160
NPAD = NW * NPT * NPASS

CHUNK = 4000
NCHUNKS = N_EDGES // CHUNK
GB = 64

BM = 2560


def _seg_max_body(y_hbm, row_hbm, col_hbm, out_hbm,
                  rowbuf, colbuf, mrow, mcol, gbuf, acc, yshared, sem):
    cid = lax.axis_index("c")
    sid = lax.axis_index("s")
    wid = sid * NC + cid

    ystripe = NPAD // NS
    for k in range(ystripe // GB):
        r = sid * ystripe + k * GB
        pltpu.sync_copy(y_hbm.at[pl.ds(r, GB)], gbuf)
        pltpu.sync_copy(gbuf, yshared.at[pl.ds(r, GB)])

    plsc.subcore_barrier()

    neg = jnp.full((16,), -jnp.inf, dtype=jnp.float32)
    zeros16 = jnp.zeros((16,), dtype=jnp.int32)

    for p in range(NPASS):
        base = p * (NW * NPT) + wid * NPT

        def init_row(i, carry):
            for d in range(D // 16):
                acc[i, pl.ds(d * 16, 16)] = neg
            return carry

        lax.fori_loop(0, NPT, init_row, 0)

        def chunk_body(ci, carry):
            off = ci * CHUNK
            pltpu.sync_copy(row_hbm.at[pl.ds(off, CHUNK)], rowbuf)
            pltpu.sync_copy(col_hbm.at[pl.ds(off, CHUNK)], colbuf)

            def filt(i, cntv):
                cv = colbuf[pl.ds(i * 16, 16)]
                rv = rowbuf[pl.ds(i * 16, 16)]
                msk = (cv >= base) & (cv < base + NPT)
                mi = lax.select(msk, jnp.ones((16,), jnp.int32),
                                jnp.zeros((16,), jnp.int32))
                cum = plsc.cumsum(mi)
                pos = cntv + cum - 1
                plsc.store_scatter(mcol, [pos], cv - base, mask=msk)
                plsc.store_scatter(mrow, [pos], rv, mask=msk)
                return cntv + plsc.all_reduce_population_count(msk)

            nv = lax.fori_loop(0, CHUNK // 16, filt,
                               jnp.zeros((16,), jnp.int32))
            n = nv[0]

            for j in range(GB // 16):
                mrow[pl.ds(n + j * 16, 16)] = zeros16

            nb = (n + GB - 1) // GB

            def batch(b, carry2):
                pltpu.async_copy(
                    yshared.at[mrow.at[pl.ds(b * GB, GB)]], gbuf, sem
                ).wait()
                ecount = jnp.minimum(jnp.int32(GB), n - b * GB)
                c0 = mcol[pl.ds(b * GB, 16)][0]

                def edge(i, c):
                    cn = mcol[pl.ds(b * GB + i + 1, 16)][0]
                    for d in range(D // 16):
                        sl = pl.ds(d * 16, 16)
                        acc[c, sl] = jnp.maximum(acc[c, sl], gbuf[i, sl])
                    return cn

                lax.fori_loop(0, ecount, edge, c0)
                return carry2

            lax.fori_loop(0, nb, batch, 0)
            return carry

        lax.fori_loop(0, NCHUNKS, chunk_body, jnp.int32(0))

        pltpu.sync_copy(acc, out_hbm.at[pl.ds(base, NPT)])


_seg_max = functools.partial(
    pl.kernel,
    out_type=jax.ShapeDtypeStruct((NPAD, D), jnp.float32),
    mesh=plsc.VectorSubcoreMesh(
        core_axis_name="c", subcore_axis_name="s", num_cores=NC, num_subcores=NS
    ),
    scratch_types=[
        pltpu.VMEM((CHUNK,), jnp.int32),
        pltpu.VMEM((CHUNK,), jnp.int32),
        pltpu.VMEM((CHUNK + GB,), jnp.int32),
        pltpu.VMEM((CHUNK + 16,), jnp.int32),
        pltpu.VMEM((GB, D), jnp.float32),
        pltpu.VMEM((NPT, D), jnp.float32),
        pltpu.VMEM_SHARED((NPAD, D), jnp.float32),
        pltpu.SemaphoreType.DMA,
    ],
    compiler_params=pltpu.CompilerParams(needs_layout_passes=False),
)(_seg_max_body)


def _mm_theta_body(x_ref, w_ref, o_ref):
    o_ref[...] = lax.dot_general(
        x_ref[...], w_ref[...], (((1,), (0,)), ((), ())),
        preferred_element_type=jnp.float32,
    )


def _final_body(m_ref, y_ref, w_ref, o_ref):
    m = m_ref[...]
    md = jnp.where(jnp.isfinite(m), m - y_ref[...], 0.0)
    o_ref[...] = jnp.maximum(
        lax.dot_general(md, w_ref[...], (((1,), (0,)), ((), ())),
                        preferred_element_type=jnp.float32),
        0.0,
    )


def kernel(x, edges, W_theta, W_phi):
    xp = jnp.pad(x, ((0, NPAD - N_NODES), (0, 0)))
    row = edges[0]
    col = edges[1]

    y = pl.pallas_call(
        _mm_theta_body,
        grid=(NPAD // BM,),
        in_specs=[
            pl.BlockSpec((BM, D), lambda i: (i, 0)),
            pl.BlockSpec((D, D), lambda i: (0, 0)),
        ],
        out_specs=pl.BlockSpec((BM, D), lambda i: (i, 0)),
        out_shape=jax.ShapeDtypeStruct((NPAD, D), jnp.float32),
    )(xp, W_theta.T)

    m = _seg_max(y, row, col)

    out = pl.pallas_call(
        _final_body,
        grid=(NPAD // BM,),
        in_specs=[
            pl.BlockSpec((BM, D), lambda i: (i, 0)),
            pl.BlockSpec((BM, D), lambda i: (i, 0)),
            pl.BlockSpec((D, D), lambda i: (0, 0)),
        ],
        out_specs=pl.BlockSpec((BM, D), lambda i: (i, 0)),
        out_shape=jax.ShapeDtypeStruct((NPAD, D), jnp.float32),
    )(m, y, W_phi.T)

    return out[:N_NODES]


# speedup vs baseline: 5.9294x; 1.1851x over previous
import functools

import jax
import jax.numpy as jnp
from jax import lax
from jax.experimental import pallas as pl
from jax.experimental.pallas import tpu as pltpu
from jax.experimental.pallas import tpu_sc as plsc

N_NODES = 10000
N_EDGES = 320000
D = 128

NC = 2
NS = 16
NW = NC * NS
NPASS = 2
NPT = 160
NPAD = NW * NPT * NPASS

CHUNK = 2000
NCHUNKS = N_EDGES // CHUNK
GB = 64

BM = 2560


def _seg_max_body(y_hbm, row_hbm, col_hbm, out_hbm,
                  rowbuf, colbuf, rowbuf2, colbuf2, mrow, mcol, gbuf, acc,
                  yshared, sem, sem2, gsem):
    cid = lax.axis_index("c")
    sid = lax.axis_index("s")
    wid = sid * NC + cid

    ystripe = NPAD // NS
    for k in range(ystripe // GB):
        r = sid * ystripe + k * GB
        pltpu.sync_copy(y_hbm.at[pl.ds(r, GB)], gbuf)
        pltpu.sync_copy(gbuf, yshared.at[pl.ds(r, GB)])

    plsc.subcore_barrier()

    neg = jnp.full((16,), -jnp.inf, dtype=jnp.float32)
    zeros16 = jnp.zeros((16,), dtype=jnp.int32)

    for p in range(NPASS):
        base = p * (NW * NPT) + wid * NPT

        def init_row(i, carry):
            for d in range(D // 16):
                acc[i, pl.ds(d * 16, 16)] = neg
            return carry

        lax.fori_loop(0, NPT, init_row, 0)

        def issue(ci, rb, cb, s):
            off = ci * CHUNK
            pltpu.async_copy(row_hbm.at[pl.ds(off, CHUNK)], rb, s)
            pltpu.async_copy(col_hbm.at[pl.ds(off, CHUNK)], cb, s)

        def drain(rb, cb, s):
            pltpu.make_async_copy(row_hbm.at[pl.ds(0, CHUNK)], rb, s).wait()
            pltpu.make_async_copy(col_hbm.at[pl.ds(0, CHUNK)], cb, s).wait()

        def process(rbuf, cbuf):
            def filt(i, cntv):
                cv = cbuf[pl.ds(i * 16, 16)]
                rv = rbuf[pl.ds(i * 16, 16)]
                msk = (cv >= base) & (cv < base + NPT)
                mi = lax.select(msk, jnp.ones((16,), jnp.int32),
                                jnp.zeros((16,), jnp.int32))
                cum = plsc.cumsum(mi)
                pos = cntv + cum - 1
                plsc.store_scatter(mcol, [pos], cv - base, mask=msk)
                plsc.store_scatter(mrow, [pos], rv, mask=msk)
                return cntv + plsc.all_reduce_population_count(msk)

            nv = lax.fori_loop(0, CHUNK // 16, filt,
                               jnp.zeros((16,), jnp.int32))
            n = nv[0]

            for j in range(GB // 16):
                mrow[pl.ds(n + j * 16, 16)] = zeros16

            nb = (n + GB - 1) // GB

            def batch(b, carry2):
                pltpu.async_copy(
                    yshared.at[mrow.at[pl.ds(b * GB, GB)]], gbuf, gsem
                ).wait()
                ecount = jnp.minimum(jnp.int32(GB), n - b * GB)
                c0 = mcol[pl.ds(b * GB, 16)][0]

                def edge(i, c):
                    cn = mcol[pl.ds(b * GB + i + 1, 16)][0]
                    for d in range(D // 16):
                        sl = pl.ds(d * 16, 16)
                        acc[c, sl] = jnp.maximum(acc[c, sl], gbuf[i, sl])
                    return cn

                lax.fori_loop(0, ecount, edge, c0)
                return carry2

            lax.fori_loop(0, nb, batch, 0)

        issue(0, rowbuf, colbuf, sem2)
        issue(1, rowbuf2, colbuf2, sem)

        def pair(j, carry):
            drain(rowbuf, colbuf, sem2)
            process(rowbuf, colbuf)

            @pl.when(2 * j + 2 < NCHUNKS)
            def _():
                issue(2 * j + 2, rowbuf, colbuf, sem2)

            drain(rowbuf2, colbuf2, sem)
            process(rowbuf2, colbuf2)

            @pl.when(2 * j + 3 < NCHUNKS)
            def _():
                issue(2 * j + 3, rowbuf2, colbuf2, sem)

            return carry

        lax.fori_loop(0, NCHUNKS // 2, pair, jnp.int32(0))

        pltpu.sync_copy(acc, out_hbm.at[pl.ds(base, NPT)])


_seg_max = functools.partial(
    pl.kernel,
    out_type=jax.ShapeDtypeStruct((NPAD, D), jnp.float32),
    mesh=plsc.VectorSubcoreMesh(
        core_axis_name="c", subcore_axis_name="s", num_cores=NC, num_subcores=NS
    ),
    scratch_types=[
        pltpu.VMEM((CHUNK,), jnp.int32),
        pltpu.VMEM((CHUNK,), jnp.int32),
        pltpu.VMEM((CHUNK,), jnp.int32),
        pltpu.VMEM((CHUNK,), jnp.int32),
        pltpu.VMEM((CHUNK + GB,), jnp.int32),
        pltpu.VMEM((CHUNK + 16,), jnp.int32),
        pltpu.VMEM((GB, D), jnp.float32),
        pltpu.VMEM((NPT, D), jnp.float32),
        pltpu.VMEM_SHARED((NPAD, D), jnp.float32),
        pltpu.SemaphoreType.DMA,
        pltpu.SemaphoreType.DMA,
        pltpu.SemaphoreType.DMA,
    ],
    compiler_params=pltpu.CompilerParams(needs_layout_passes=False),
)(_seg_max_body)


def _mm_theta_body(x_ref, w_ref, o_ref):
    o_ref[...] = lax.dot_general(
        x_ref[...], w_ref[...], (((1,), (0,)), ((), ())),
        preferred_element_type=jnp.float32,
    )


def _final_body(m_ref, y_ref, w_ref, o_ref):
    m = m_ref[...]
    md = jnp.where(jnp.isfinite(m), m - y_ref[...], 0.0)
    o_ref[...] = jnp.maximum(
        lax.dot_general(md, w_ref[...], (((1,), (0,)), ((), ())),
                        preferred_element_type=jnp.float32),
        0.0,
    )


def kernel(x, edges, W_theta, W_phi):
    xp = jnp.pad(x, ((0, NPAD - N_NODES), (0, 0)))
    row = edges[0]
    col = edges[1]

    y = pl.pallas_call(
        _mm_theta_body,
        grid=(NPAD // BM,),
        in_specs=[
            pl.BlockSpec((BM, D), lambda i: (i, 0)),
            pl.BlockSpec((D, D), lambda i: (0, 0)),
        ],
        out_specs=pl.BlockSpec((BM, D), lambda i: (i, 0)),
        out_shape=jax.ShapeDtypeStruct((NPAD, D), jnp.float32),
    )(xp, W_theta.T)

    m = _seg_max(y, row, col)

    out = pl.pallas_call(
        _final_body,
        grid=(NPAD // BM,),
        in_specs=[
            pl.BlockSpec((BM, D), lambda i: (i, 0)),
            pl.BlockSpec((BM, D), lambda i: (i, 0)),
            pl.BlockSpec((D, D), lambda i: (0, 0)),
        ],
        out_specs=pl.BlockSpec((BM, D), lambda i: (i, 0)),
        out_shape=jax.ShapeDtypeStruct((NPAD, D), jnp.float32),
    )(m, y, W_phi.T)

    return out[:N_NODES]
